# baseline (speedup 1.0000x reference)
import functools

import jax
import jax.numpy as jnp
import numpy as np
from jax.experimental import pallas as pl
from jax.experimental.pallas import tpu as pltpu

_N_SCALAR = 32
_N_VEC = 16
_N_TEN = 8
_DIM = _N_SCALAR + 3 * _N_VEC + 5 * _N_TEN
_NG = _N_VEC + _N_TEN
_EPS = 1e-05
_CB = 4096
_PREC = jax.lax.Precision.DEFAULT


def _group_cols():
    pairs = []
    for g in range(_N_VEC):
        for k in range(3):
            pairs.append((_N_SCALAR + 3 * g + k, g))
    for t in range(_N_TEN):
        for k in range(5):
            pairs.append((_N_SCALAR + 3 * _N_VEC + 5 * t + k, _N_VEC + t))
    return pairs


def _const_mats():
    vzt = np.zeros((128, _DIM), np.float32)
    for c, g in _group_cols():
        vzt[g, c] = 1.0
    vzt[24, :_N_SCALAR] = 1.0 / _N_SCALAR
    vzt[25, :_N_SCALAR] = 1.0 / _N_SCALAR
    vxt = np.zeros((128, _DIM), np.float32)
    vxt[24, :_N_SCALAR] = 1.0 / _N_SCALAR
    vxt[25, :_N_SCALAR] = 1.0 / _N_SCALAR
    ev = np.zeros((128, 1), np.float32)
    ev[24, 0] = _EPS
    ev[25, 0] = _EPS
    m25 = np.zeros((128, 1), np.float32)
    m25[25, 0] = 1.0
    m26 = np.zeros((128, 1), np.float32)
    m26[26, 0] = 1.0
    gct = np.zeros((256, 128), np.float32)
    for c, g in _group_cols():
        gct[c, g] = 1.0
    return (jnp.asarray(vzt), jnp.asarray(vxt), jnp.asarray(ev),
            jnp.asarray(m25), jnp.asarray(m26), jnp.asarray(gct))


def _body(x_ref, w_ref, b_ref, vzt_ref, vxt_ref, gct_ref, ev_ref, m25_ref,
          m26_ref, o_ref, acc_ref, *, n_rows):
    p = pl.program_id(0)
    i = pl.program_id(1)
    x = x_ref[...]

    @pl.when(p == 0)
    def _phase0():
        norm2 = jax.lax.dot(vzt_ref[...], x * x, precision=_PREC,
                            preferred_element_type=jnp.float32)
        n2c = jnp.maximum(norm2, 1e-12)
        norm = jnp.maximum(norm2 * jax.lax.rsqrt(n2c), 1e-06)
        cols = jax.lax.broadcasted_iota(jnp.int32, (1, _CB), 1) + i * _CB
        norm = jnp.where(cols < n_rows, norm, 0.0)
        psum = jnp.sum(norm, axis=1, keepdims=True)

        @pl.when(i == 0)
        def _():
            acc_ref[...] = jnp.zeros_like(acc_ref)

        acc_ref[...] += psum

    @pl.when(p == 1)
    def _phase1():
        mz = jax.lax.dot(vzt_ref[...], x * x, precision=_PREC,
                         preferred_element_type=jnp.float32)
        mx = jax.lax.dot(vxt_ref[...], x, precision=_PREC,
                         preferred_element_type=jnp.float32)
        t = jnp.maximum(mz - mx * mx + ev_ref[...], 1e-12)
        rall = jax.lax.rsqrt(t)
        row1 = jax.lax.broadcasted_iota(jnp.int32, (128, 1), 0)
        coef = jnp.where(row1 < _NG, acc_ref[...] * (1.0 / n_rows),
                         jnp.where(row1 == _NG, 1.0, 0.0))
        s = rall * coef + (mx * rall) * m25_ref[...] + m26_ref[...]
        row = jax.lax.broadcasted_iota(jnp.int32, (256, 128), 0)
        col = jax.lax.broadcasted_iota(jnp.int32, (256, 128), 1)
        zpad = jnp.zeros((128 - _N_SCALAR, 1), jnp.float32)
        w128 = jnp.concatenate([w_ref[...], zpad], axis=0)
        b128 = jnp.concatenate([b_ref[...], zpad], axis=0)
        w256 = jnp.concatenate([w128, w128], axis=0)
        b256 = jnp.concatenate([b128, b128], axis=0)
        in_a = row < 128
        g = gct_ref[...]
        g = jnp.where((col == 24) & in_a, w256, g)
        g = jnp.where((col == 25) & ~in_a, -w256, g)
        g = jnp.where((col == 26) & ~in_a, b256, g)
        ab = jax.lax.dot(g, s, precision=_PREC,
                         preferred_element_type=jnp.float32)
        o_ref[...] = x * ab[:_DIM, :] + ab[128:128 + _DIM, :]


def kernel(x, weight, bias):
    n = x.shape[0]
    nb = pl.cdiv(n, _CB)
    vzt, vxt, ev, m25, m26, gct = _const_mats()
    xt = x.T
    w2 = weight.reshape(_N_SCALAR, 1)
    b2 = bias.reshape(_N_SCALAR, 1)

    out_t = pl.pallas_call(
        functools.partial(_body, n_rows=n),
        grid=(2, nb),
        in_specs=[
            pl.BlockSpec((_DIM, _CB), lambda p, i: (0, i)),
            pl.BlockSpec((_N_SCALAR, 1), lambda p, i: (0, 0)),
            pl.BlockSpec((_N_SCALAR, 1), lambda p, i: (0, 0)),
            pl.BlockSpec((128, _DIM), lambda p, i: (0, 0)),
            pl.BlockSpec((128, _DIM), lambda p, i: (0, 0)),
            pl.BlockSpec((256, 128), lambda p, i: (0, 0)),
            pl.BlockSpec((128, 1), lambda p, i: (0, 0)),
            pl.BlockSpec((128, 1), lambda p, i: (0, 0)),
            pl.BlockSpec((128, 1), lambda p, i: (0, 0)),
        ],
        out_specs=pl.BlockSpec((_DIM, _CB), lambda p, i: (0, p * i)),
        out_shape=jax.ShapeDtypeStruct((_DIM, n), jnp.float32),
        scratch_shapes=[pltpu.VMEM((128, 1), jnp.float32)],
        compiler_params=pltpu.CompilerParams(
            dimension_semantics=("arbitrary", "arbitrary")),
    )(xt, w2, b2, vzt, vxt, gct, ev, m25, m26)
    return out_t.T


# speedup vs baseline: 1.0963x; 1.0963x over previous
import functools

import jax
import jax.numpy as jnp
import numpy as np
from jax.experimental import pallas as pl
from jax.experimental.pallas import tpu as pltpu

_N_SCALAR = 32
_N_VEC = 16
_N_TEN = 8
_DIM = _N_SCALAR + 3 * _N_VEC + 5 * _N_TEN
_NG = _N_VEC + _N_TEN
_EPS = 1e-05
_CB = 4096
_PREC = jax.lax.Precision.DEFAULT


def _group_cols():
    pairs = []
    for g in range(_N_VEC):
        for k in range(3):
            pairs.append((_N_SCALAR + 3 * g + k, g))
    for t in range(_N_TEN):
        for k in range(5):
            pairs.append((_N_SCALAR + 3 * _N_VEC + 5 * t + k, _N_VEC + t))
    return pairs


def _const_mats():
    vzt = np.zeros((128, _DIM), np.float32)
    for c, g in _group_cols():
        vzt[g, c] = 1.0
    vzt[24, :_N_SCALAR] = 1.0 / _N_SCALAR
    vzt[25, :_N_SCALAR] = 1.0 / _N_SCALAR
    vxt = np.zeros((128, _DIM), np.float32)
    vxt[24, :_N_SCALAR] = 1.0 / _N_SCALAR
    vxt[25, :_N_SCALAR] = 1.0 / _N_SCALAR
    ev = np.zeros((128, 1), np.float32)
    ev[24, 0] = _EPS
    ev[25, 0] = _EPS
    m25 = np.zeros((128, 1), np.float32)
    m25[25, 0] = 1.0
    m26 = np.zeros((128, 1), np.float32)
    m26[26, 0] = 1.0
    gct = np.zeros((256, 128), np.float32)
    for c, g in _group_cols():
        gct[c, g] = 1.0
    return (jnp.asarray(vzt), jnp.asarray(vxt), jnp.asarray(ev),
            jnp.asarray(m25), jnp.asarray(m26), jnp.asarray(gct))


def _body(x_ref, w_ref, b_ref, vzt_ref, vxt_ref, gct_ref, ev_ref, m25_ref,
          m26_ref, o_ref, xbig_ref, acc_ref, *, n_rows, nb):
    p = pl.program_id(0)
    i = pl.program_id(1)

    @pl.when(p == 0)
    def _phase0():
        x = x_ref[...]
        xbig_ref[:, pl.ds(pl.multiple_of(i * _CB, _CB), _CB)] = x
        norm2 = jax.lax.dot(vzt_ref[...], x * x, precision=_PREC,
                            preferred_element_type=jnp.float32)
        n2c = jnp.maximum(norm2, 1e-12)
        norm = jnp.maximum(norm2 * jax.lax.rsqrt(n2c), 1e-06)
        cols = jax.lax.broadcasted_iota(jnp.int32, (1, _CB), 1) + i * _CB
        norm = jnp.where(cols < n_rows, norm, 0.0)
        psum = jnp.sum(norm, axis=1, keepdims=True)

        @pl.when(i == 0)
        def _():
            acc_ref[...] = jnp.zeros_like(acc_ref)

        acc_ref[...] += psum

    @pl.when(p == 1)
    def _phase1():
        x = xbig_ref[:, pl.ds(pl.multiple_of(i * _CB, _CB), _CB)]
        mz = jax.lax.dot(vzt_ref[...], x * x, precision=_PREC,
                         preferred_element_type=jnp.float32)
        mx = jax.lax.dot(vxt_ref[...], x, precision=_PREC,
                         preferred_element_type=jnp.float32)
        t = jnp.maximum(mz - mx * mx + ev_ref[...], 1e-12)
        rall = jax.lax.rsqrt(t)
        row1 = jax.lax.broadcasted_iota(jnp.int32, (128, 1), 0)
        coef = jnp.where(row1 < _NG, acc_ref[...] * (1.0 / n_rows),
                         jnp.where(row1 == _NG, 1.0, 0.0))
        s = rall * coef + (mx * rall) * m25_ref[...] + m26_ref[...]
        row = jax.lax.broadcasted_iota(jnp.int32, (256, 128), 0)
        col = jax.lax.broadcasted_iota(jnp.int32, (256, 128), 1)
        zpad = jnp.zeros((128 - _N_SCALAR, 1), jnp.float32)
        w128 = jnp.concatenate([w_ref[...], zpad], axis=0)
        b128 = jnp.concatenate([b_ref[...], zpad], axis=0)
        w256 = jnp.concatenate([w128, w128], axis=0)
        b256 = jnp.concatenate([b128, b128], axis=0)
        in_a = row < 128
        g = gct_ref[...]
        g = jnp.where((col == 24) & in_a, w256, g)
        g = jnp.where((col == 25) & ~in_a, -w256, g)
        g = jnp.where((col == 26) & ~in_a, b256, g)
        ab = jax.lax.dot(g, s, precision=_PREC,
                         preferred_element_type=jnp.float32)
        o_ref[...] = x * ab[:_DIM, :] + ab[128:128 + _DIM, :]


def kernel(x, weight, bias):
    n = x.shape[0]
    nb = pl.cdiv(n, _CB)
    vzt, vxt, ev, m25, m26, gct = _const_mats()
    xt = x.T
    w2 = weight.reshape(_N_SCALAR, 1)
    b2 = bias.reshape(_N_SCALAR, 1)

    out_t = pl.pallas_call(
        functools.partial(_body, n_rows=n, nb=nb),
        grid=(2, nb),
        in_specs=[
            pl.BlockSpec((_DIM, _CB), lambda p, i: (0, i + p * (nb - 1 - i))),
            pl.BlockSpec((_N_SCALAR, 1), lambda p, i: (0, 0)),
            pl.BlockSpec((_N_SCALAR, 1), lambda p, i: (0, 0)),
            pl.BlockSpec((128, _DIM), lambda p, i: (0, 0)),
            pl.BlockSpec((128, _DIM), lambda p, i: (0, 0)),
            pl.BlockSpec((256, 128), lambda p, i: (0, 0)),
            pl.BlockSpec((128, 1), lambda p, i: (0, 0)),
            pl.BlockSpec((128, 1), lambda p, i: (0, 0)),
            pl.BlockSpec((128, 1), lambda p, i: (0, 0)),
        ],
        out_specs=pl.BlockSpec((_DIM, _CB), lambda p, i: (0, p * i)),
        out_shape=jax.ShapeDtypeStruct((_DIM, n), jnp.float32),
        scratch_shapes=[
            pltpu.VMEM((_DIM, nb * _CB), jnp.float32),
            pltpu.VMEM((128, 1), jnp.float32),
        ],
        compiler_params=pltpu.CompilerParams(
            dimension_semantics=("arbitrary", "arbitrary")),
    )(xt, w2, b2, vzt, vxt, gct, ev, m25, m26)
    return out_t.T


# speedup vs baseline: 1.1801x; 1.0764x over previous
import functools

import jax
import jax.numpy as jnp
import numpy as np
from jax.experimental import pallas as pl
from jax.experimental.pallas import tpu as pltpu

_N_SCALAR = 32
_N_VEC = 16
_N_TEN = 8
_DIM = _N_SCALAR + 3 * _N_VEC + 5 * _N_TEN
_NG = _N_VEC + _N_TEN
_EPS = 1e-05
_CB = 8192
_PREC = jax.lax.Precision.DEFAULT


def _group_cols():
    pairs = []
    for g in range(_N_VEC):
        for k in range(3):
            pairs.append((_N_SCALAR + 3 * g + k, g))
    for t in range(_N_TEN):
        for k in range(5):
            pairs.append((_N_SCALAR + 3 * _N_VEC + 5 * t + k, _N_VEC + t))
    return pairs


def _const_mats():
    vzt = np.zeros((128, _DIM), np.float32)
    for c, g in _group_cols():
        vzt[g, c] = 1.0
    vzt[24, :_N_SCALAR] = 1.0 / _N_SCALAR
    vzt[25, :_N_SCALAR] = 1.0 / _N_SCALAR
    vxt = np.zeros((128, _DIM), np.float32)
    vxt[24, :_N_SCALAR] = 1.0 / _N_SCALAR
    vxt[25, :_N_SCALAR] = 1.0 / _N_SCALAR
    ev = np.zeros((128, 1), np.float32)
    ev[24, 0] = _EPS
    ev[25, 0] = _EPS
    m25 = np.zeros((128, 1), np.float32)
    m25[25, 0] = 1.0
    m26 = np.zeros((128, 1), np.float32)
    m26[26, 0] = 1.0
    gct = np.zeros((256, 128), np.float32)
    for c, g in _group_cols():
        gct[c, g] = 1.0
    return (jnp.asarray(vzt), jnp.asarray(vxt), jnp.asarray(ev),
            jnp.asarray(m25), jnp.asarray(m26), jnp.asarray(gct))


def _body(x_ref, w_ref, b_ref, vzt_ref, vxt_ref, gct_ref, ev_ref, m25_ref,
          m26_ref, o_ref, xbig_ref, acc_ref, *, n_rows, nb):
    p = pl.program_id(0)
    i = pl.program_id(1)

    @pl.when(p == 0)
    def _phase0():
        x = x_ref[...]
        xbig_ref[:, pl.ds(pl.multiple_of(i * _CB, _CB), _CB)] = (
            x.astype(jnp.bfloat16))
        norm2 = jax.lax.dot(vzt_ref[...], x * x, precision=_PREC,
                            preferred_element_type=jnp.float32)
        n2c = jnp.maximum(norm2, 1e-12)
        norm = jnp.maximum(norm2 * jax.lax.rsqrt(n2c), 1e-06)
        cols = jax.lax.broadcasted_iota(jnp.int32, (1, _CB), 1) + i * _CB
        norm = jnp.where(cols < n_rows, norm, 0.0)
        psum = jnp.sum(norm, axis=1, keepdims=True)

        @pl.when(i == 0)
        def _():
            acc_ref[...] = jnp.zeros_like(acc_ref)

        acc_ref[...] += psum

    @pl.when(p == 1)
    def _phase1():
        x = xbig_ref[:, pl.ds(pl.multiple_of(i * _CB, _CB), _CB)].astype(
            jnp.float32)
        mz = jax.lax.dot(vzt_ref[...], x * x, precision=_PREC,
                         preferred_element_type=jnp.float32)
        mx = jax.lax.dot(vxt_ref[...], x, precision=_PREC,
                         preferred_element_type=jnp.float32)
        t = jnp.maximum(mz - mx * mx + ev_ref[...], 1e-12)
        rall = jax.lax.rsqrt(t)
        row1 = jax.lax.broadcasted_iota(jnp.int32, (128, 1), 0)
        coef = jnp.where(row1 < _NG, acc_ref[...] * (1.0 / n_rows),
                         jnp.where(row1 == _NG, 1.0, 0.0))
        s = rall * coef + (mx * rall) * m25_ref[...] + m26_ref[...]
        row = jax.lax.broadcasted_iota(jnp.int32, (256, 128), 0)
        col = jax.lax.broadcasted_iota(jnp.int32, (256, 128), 1)
        zpad = jnp.zeros((128 - _N_SCALAR, 1), jnp.float32)
        w128 = jnp.concatenate([w_ref[...], zpad], axis=0)
        b128 = jnp.concatenate([b_ref[...], zpad], axis=0)
        w256 = jnp.concatenate([w128, w128], axis=0)
        b256 = jnp.concatenate([b128, b128], axis=0)
        in_a = row < 128
        g = gct_ref[...]
        g = jnp.where((col == 24) & in_a, w256, g)
        g = jnp.where((col == 25) & ~in_a, -w256, g)
        g = jnp.where((col == 26) & ~in_a, b256, g)
        ab = jax.lax.dot(g, s, precision=_PREC,
                         preferred_element_type=jnp.float32)
        o_ref[...] = x * ab[:_DIM, :] + ab[128:128 + _DIM, :]


def kernel(x, weight, bias):
    n = x.shape[0]
    nb = pl.cdiv(n, _CB)
    vzt, vxt, ev, m25, m26, gct = _const_mats()
    xt = x.T
    w2 = weight.reshape(_N_SCALAR, 1)
    b2 = bias.reshape(_N_SCALAR, 1)

    out_t = pl.pallas_call(
        functools.partial(_body, n_rows=n, nb=nb),
        grid=(2, nb),
        in_specs=[
            pl.BlockSpec((_DIM, _CB), lambda p, i: (0, i + p * (nb - 1 - i))),
            pl.BlockSpec((_N_SCALAR, 1), lambda p, i: (0, 0)),
            pl.BlockSpec((_N_SCALAR, 1), lambda p, i: (0, 0)),
            pl.BlockSpec((128, _DIM), lambda p, i: (0, 0)),
            pl.BlockSpec((128, _DIM), lambda p, i: (0, 0)),
            pl.BlockSpec((256, 128), lambda p, i: (0, 0)),
            pl.BlockSpec((128, 1), lambda p, i: (0, 0)),
            pl.BlockSpec((128, 1), lambda p, i: (0, 0)),
            pl.BlockSpec((128, 1), lambda p, i: (0, 0)),
        ],
        out_specs=pl.BlockSpec((_DIM, _CB), lambda p, i: (0, p * i)),
        out_shape=jax.ShapeDtypeStruct((_DIM, n), jnp.float32),
        scratch_shapes=[
            pltpu.VMEM((_DIM, nb * _CB), jnp.bfloat16),
            pltpu.VMEM((128, 1), jnp.float32),
        ],
        compiler_params=pltpu.CompilerParams(
            dimension_semantics=("arbitrary", "arbitrary")),
    )(xt, w2, b2, vzt, vxt, gct, ev, m25, m26)
    return out_t.T


# speedup vs baseline: 1.8062x; 1.5306x over previous
import functools

import jax
import jax.numpy as jnp
import numpy as np
from jax.experimental import pallas as pl
from jax.experimental.pallas import tpu as pltpu

_N_SCALAR = 32
_N_VEC = 16
_N_TEN = 8
_DIM = _N_SCALAR + 3 * _N_VEC + 5 * _N_TEN
_NG = _N_VEC + _N_TEN
_EPS = 1e-05
_CB = 8192
_PREC = jax.lax.Precision.DEFAULT


def _group_cols():
    pairs = []
    for g in range(_N_VEC):
        for k in range(3):
            pairs.append((_N_SCALAR + 3 * g + k, g))
    for t in range(_N_TEN):
        for k in range(5):
            pairs.append((_N_SCALAR + 3 * _N_VEC + 5 * t + k, _N_VEC + t))
    return pairs


def _const_mats():
    vzt = np.zeros((32, _DIM), np.float32)
    for c, g in _group_cols():
        vzt[g, c] = 1.0
    vzt[24, :_N_SCALAR] = 1.0 / _N_SCALAR
    vzt[25, :_N_SCALAR] = 1.0 / _N_SCALAR
    vxt = np.zeros((32, _DIM), np.float32)
    vxt[24, :_N_SCALAR] = 1.0 / _N_SCALAR
    vxt[25, :_N_SCALAR] = 1.0 / _N_SCALAR
    ev = np.zeros((32, 1), np.float32)
    ev[24, 0] = _EPS
    ev[25, 0] = _EPS
    m25 = np.zeros((32, 1), np.float32)
    m25[25, 0] = 1.0
    m26 = np.zeros((32, 1), np.float32)
    m26[26, 0] = 1.0
    gct = np.zeros((256, 32), np.float32)
    for c, g in _group_cols():
        gct[c, g] = 1.0
    return (jnp.asarray(vzt, dtype=jnp.bfloat16),
            jnp.asarray(vxt, dtype=jnp.bfloat16), jnp.asarray(ev),
            jnp.asarray(m25), jnp.asarray(m26), jnp.asarray(gct))


def _body(x_ref, w_ref, b_ref, vzt_ref, vxt_ref, gct_ref, ev_ref, m25_ref,
          m26_ref, o_ref, xbig_ref, acc_ref, *, n_rows, nb):
    p = pl.program_id(0)
    i = pl.program_id(1)

    @pl.when(p == 0)
    def _phase0():
        xb = x_ref[...].astype(jnp.bfloat16)
        xbig_ref[:, pl.ds(pl.multiple_of(i * _CB, _CB), _CB)] = xb
        norm2 = jax.lax.dot(vzt_ref[...], xb * xb, precision=_PREC,
                            preferred_element_type=jnp.float32)
        n2c = jnp.maximum(norm2, 1e-12)
        norm = jnp.maximum(norm2 * jax.lax.rsqrt(n2c), 1e-06)
        cols = jax.lax.broadcasted_iota(jnp.int32, (1, _CB), 1) + i * _CB
        norm = jnp.where(cols < n_rows, norm, 0.0)
        psum = jnp.sum(norm, axis=1, keepdims=True)

        @pl.when(i == 0)
        def _():
            acc_ref[...] = jnp.zeros_like(acc_ref)

        acc_ref[...] += psum

    @pl.when(p == 1)
    def _phase1():
        xb = xbig_ref[:, pl.ds(pl.multiple_of(i * _CB, _CB), _CB)]
        mz = jax.lax.dot(vzt_ref[...], xb * xb, precision=_PREC,
                         preferred_element_type=jnp.float32)
        mx = jax.lax.dot(vxt_ref[...], xb, precision=_PREC,
                         preferred_element_type=jnp.float32)
        t = jnp.maximum(mz - mx * mx + ev_ref[...], 1e-12)
        rall = jax.lax.rsqrt(t)
        row1 = jax.lax.broadcasted_iota(jnp.int32, (32, 1), 0)
        coef = jnp.where(row1 < _NG, acc_ref[...] * (1.0 / n_rows),
                         jnp.where(row1 == _NG, 1.0, 0.0))
        s = rall * coef + (mx * rall) * m25_ref[...] + m26_ref[...]
        row = jax.lax.broadcasted_iota(jnp.int32, (256, 32), 0)
        col = jax.lax.broadcasted_iota(jnp.int32, (256, 32), 1)
        w = w_ref[...]
        b = b_ref[...]
        zpad = jnp.zeros((128 - _N_SCALAR, 1), jnp.float32)
        w128 = jnp.concatenate([w, zpad], axis=0)
        w256 = jnp.concatenate([w128, w128], axis=0)
        b256 = jnp.concatenate([w128 * 0.0, jnp.concatenate(
            [b, zpad], axis=0)], axis=0)
        in_a = row < 128
        g = gct_ref[...]
        g = jnp.where((col == 24) & in_a, w256, g)
        g = jnp.where((col == 25) & ~in_a, -w256, g)
        g = jnp.where((col == 26) & ~in_a, b256, g)
        ab = jax.lax.dot(g, s, precision=_PREC,
                         preferred_element_type=jnp.float32)
        x = xb.astype(jnp.float32)
        o_ref[...] = x * ab[:_DIM, :] + ab[128:128 + _DIM, :]


def kernel(x, weight, bias):
    n = x.shape[0]
    nb = pl.cdiv(n, _CB)
    vzt, vxt, ev, m25, m26, gct = _const_mats()
    xt = x.T
    w2 = weight.reshape(_N_SCALAR, 1)
    b2 = bias.reshape(_N_SCALAR, 1)

    out_t = pl.pallas_call(
        functools.partial(_body, n_rows=n, nb=nb),
        grid=(2, nb),
        in_specs=[
            pl.BlockSpec((_DIM, _CB), lambda p, i: (0, i + p * (nb - 1 - i))),
            pl.BlockSpec((_N_SCALAR, 1), lambda p, i: (0, 0)),
            pl.BlockSpec((_N_SCALAR, 1), lambda p, i: (0, 0)),
            pl.BlockSpec((32, _DIM), lambda p, i: (0, 0)),
            pl.BlockSpec((32, _DIM), lambda p, i: (0, 0)),
            pl.BlockSpec((256, 32), lambda p, i: (0, 0)),
            pl.BlockSpec((32, 1), lambda p, i: (0, 0)),
            pl.BlockSpec((32, 1), lambda p, i: (0, 0)),
            pl.BlockSpec((32, 1), lambda p, i: (0, 0)),
        ],
        out_specs=pl.BlockSpec((_DIM, _CB), lambda p, i: (0, p * i)),
        out_shape=jax.ShapeDtypeStruct((_DIM, n), jnp.float32),
        scratch_shapes=[
            pltpu.VMEM((_DIM, nb * _CB), jnp.bfloat16),
            pltpu.VMEM((32, 1), jnp.float32),
        ],
        compiler_params=pltpu.CompilerParams(
            dimension_semantics=("arbitrary", "arbitrary")),
    )(xt, w2, b2, vzt, vxt, gct, ev, m25, m26)
    return out_t.T


# speedup vs baseline: 1.8998x; 1.0518x over previous
import functools

import jax
import jax.numpy as jnp
import numpy as np
from jax.experimental import pallas as pl
from jax.experimental.pallas import tpu as pltpu

_N_SCALAR = 32
_N_VEC = 16
_N_TEN = 8
_DIM = _N_SCALAR + 3 * _N_VEC + 5 * _N_TEN
_NG = _N_VEC + _N_TEN
_EPS = 1e-05
_CB = 8192
_PREC = jax.lax.Precision.DEFAULT


def _group_cols():
    pairs = []
    for g in range(_N_VEC):
        for k in range(3):
            pairs.append((_N_SCALAR + 3 * g + k, g))
    for t in range(_N_TEN):
        for k in range(5):
            pairs.append((_N_SCALAR + 3 * _N_VEC + 5 * t + k, _N_VEC + t))
    return pairs


def _const_mats():
    vzt = np.zeros((32, _DIM), np.float32)
    for c, g in _group_cols():
        vzt[g, c] = 1.0
    vzt[24, :_N_SCALAR] = 1.0 / _N_SCALAR
    vzt[25, :_N_SCALAR] = 1.0 / _N_SCALAR
    vxt = np.zeros((32, _DIM), np.float32)
    vxt[24, :_N_SCALAR] = 1.0 / _N_SCALAR
    vxt[25, :_N_SCALAR] = 1.0 / _N_SCALAR
    ev = np.zeros((32, 1), np.float32)
    ev[24, 0] = _EPS
    ev[25, 0] = _EPS
    m25 = np.zeros((32, 1), np.float32)
    m25[25, 0] = 1.0
    m26 = np.zeros((32, 1), np.float32)
    m26[26, 0] = 1.0
    gct = np.zeros((256, 32), np.float32)
    for c, g in _group_cols():
        gct[c, g] = 1.0
    return (jnp.asarray(vzt, dtype=jnp.bfloat16),
            jnp.asarray(vxt, dtype=jnp.bfloat16), jnp.asarray(ev),
            jnp.asarray(m25), jnp.asarray(m26), jnp.asarray(gct))


def _body(x_ref, w_ref, b_ref, vzt_ref, vxt_ref, gct_ref, ev_ref, m25_ref,
          m26_ref, o_ref, xbig_ref, acc_ref, *, n_rows, nb):
    p = pl.program_id(0)
    i = pl.program_id(1)

    @pl.when(p == 0)
    def _phase0():
        xb = x_ref[...].astype(jnp.bfloat16)
        xbig_ref[:, pl.ds(pl.multiple_of(i * _CB, _CB), _CB)] = xb
        norm2 = jax.lax.dot(vzt_ref[...], xb * xb, precision=_PREC,
                            preferred_element_type=jnp.float32)
        n2c = jnp.maximum(norm2, 1e-12)
        norm = jnp.maximum(norm2 * jax.lax.rsqrt(n2c), 1e-06)
        cols = jax.lax.broadcasted_iota(jnp.int32, (1, _CB), 1) + i * _CB
        norm = jnp.where(cols < n_rows, norm, 0.0)
        psum = jnp.sum(norm, axis=1, keepdims=True)

        @pl.when(i == 0)
        def _():
            acc_ref[...] = jnp.zeros_like(acc_ref)

        acc_ref[...] += psum

    @pl.when(p == 1)
    def _phase1():
        xb = xbig_ref[:, pl.ds(pl.multiple_of(i * _CB, _CB), _CB)]
        mz = jax.lax.dot(vzt_ref[...], xb * xb, precision=_PREC,
                         preferred_element_type=jnp.float32)
        mx = jax.lax.dot(vxt_ref[...], xb, precision=_PREC,
                         preferred_element_type=jnp.float32)
        t = jnp.maximum(mz - mx * mx + ev_ref[...], 1e-12)
        rall = jax.lax.rsqrt(t)
        row1 = jax.lax.broadcasted_iota(jnp.int32, (32, 1), 0)
        coef = jnp.where(row1 < _NG, acc_ref[...] * (1.0 / n_rows),
                         jnp.where(row1 == _NG, 1.0, 0.0))
        s = rall * coef + (mx * rall) * m25_ref[...] + m26_ref[...]
        row = jax.lax.broadcasted_iota(jnp.int32, (256, 32), 0)
        col = jax.lax.broadcasted_iota(jnp.int32, (256, 32), 1)
        w = w_ref[...].T
        b = b_ref[...].T
        zpad = jnp.zeros((128 - _N_SCALAR, 1), jnp.float32)
        w128 = jnp.concatenate([w, zpad], axis=0)
        w256 = jnp.concatenate([w128, w128], axis=0)
        b256 = jnp.concatenate([w128 * 0.0, jnp.concatenate(
            [b, zpad], axis=0)], axis=0)
        in_a = row < 128
        g = gct_ref[...]
        g = jnp.where((col == 24) & in_a, w256, g)
        g = jnp.where((col == 25) & ~in_a, -w256, g)
        g = jnp.where((col == 26) & ~in_a, b256, g)
        ab = jax.lax.dot(g, s, precision=_PREC,
                         preferred_element_type=jnp.float32)
        x = xb.astype(jnp.float32)
        o_ref[...] = x * ab[:_DIM, :] + ab[128:128 + _DIM, :]


def kernel(x, weight, bias):
    n = x.shape[0]
    nb = pl.cdiv(n, _CB)
    vzt, vxt, ev, m25, m26, gct = _const_mats()
    xt = x.T
    w2 = weight.reshape(1, _N_SCALAR)
    b2 = bias.reshape(1, _N_SCALAR)

    out_t = pl.pallas_call(
        functools.partial(_body, n_rows=n, nb=nb),
        grid=(2, nb),
        in_specs=[
            pl.BlockSpec((_DIM, _CB), lambda p, i: (0, i + p * (nb - 1 - i))),
            pl.BlockSpec((1, _N_SCALAR), lambda p, i: (0, 0)),
            pl.BlockSpec((1, _N_SCALAR), lambda p, i: (0, 0)),
            pl.BlockSpec((32, _DIM), lambda p, i: (0, 0)),
            pl.BlockSpec((32, _DIM), lambda p, i: (0, 0)),
            pl.BlockSpec((256, 32), lambda p, i: (0, 0)),
            pl.BlockSpec((32, 1), lambda p, i: (0, 0)),
            pl.BlockSpec((32, 1), lambda p, i: (0, 0)),
            pl.BlockSpec((32, 1), lambda p, i: (0, 0)),
        ],
        out_specs=pl.BlockSpec((_DIM, _CB), lambda p, i: (0, p * i)),
        out_shape=jax.ShapeDtypeStruct((_DIM, n), jnp.float32),
        scratch_shapes=[
            pltpu.VMEM((_DIM, nb * _CB), jnp.bfloat16),
            pltpu.VMEM((32, 1), jnp.float32),
        ],
        compiler_params=pltpu.CompilerParams(
            dimension_semantics=("arbitrary", "arbitrary")),
    )(xt, w2, b2, vzt, vxt, gct, ev, m25, m26)
    return out_t.T
